# baseline (speedup 1.0000x reference)
import functools

import jax
import jax.numpy as jnp
from jax import lax
from jax.experimental import pallas as pl
from jax.experimental.pallas import tpu as pltpu
from jax.experimental.pallas import tpu_sc as plsc

N_ATOMS = 50000
N_BONDS = 110000
N_TREE = 20000
H = 128
ATOM_FDIM = 35
BOND_FDIM = 5
MAX_NB = 15
DEPTH = 3
N_MOLS = 1000
LE = 50

_NC = 2
_NS = 16
_NW = _NC * _NS
_RS = 8

NBP = 110592
NAP = 50176
TREE_PAD = 20480
TBL = TREE_PAD + NBP + 1024
_BB = 1024


@functools.lru_cache(maxsize=None)
def _make_gather_sum(n_rows: int):
    rpw = n_rows // _NW
    nsteps = rpw // _RS
    gwidth = _RS * MAX_NB
    wlen = nsteps * gwidth

    mesh = plsc.VectorSubcoreMesh(
        core_axis_name="c", subcore_axis_name="s",
        num_cores=_NC, num_subcores=_NS)

    def body(table, idxs, out, idx_all, buf0, buf1, ob0, ob1,
             gsem0, gsem1, osem0, osem1):
        cid = lax.axis_index("c")
        sid = lax.axis_index("s")
        wid = sid * _NC + cid
        base = wid * rpw

        pltpu.sync_copy(idxs.at[pl.ds(wid * wlen, wlen)], idx_all)
        pltpu.async_copy(table.at[idx_all.at[pl.ds(0, gwidth)]], buf0, gsem0)

        def halfstep(st, buf, gsem, nbuf, ngsem, ob, osem):
            @pl.when(st + 1 < nsteps)
            def _():
                pltpu.async_copy(
                    table.at[idx_all.at[pl.ds((st + 1) * gwidth, gwidth)]],
                    nbuf, ngsem)

            pltpu.make_async_copy(
                table.at[idx_all.at[pl.ds(st * gwidth, gwidth)]],
                buf, gsem).wait()

            @pl.when(st >= 2)
            def _():
                pltpu.make_async_copy(
                    ob, out.at[pl.ds(base, _RS), :], osem).wait()

            def row(r, carry):
                rb = r * MAX_NB
                mask = jnp.uint32(0xFFFF0000)
                sh = jnp.uint32(16)

                def split(j):
                    xs = [buf[rb + j, pl.ds(k * 16, 16)] for k in range(4)]
                    los = [lax.bitcast_convert_type(x << sh, jnp.float32)
                           for x in xs]
                    his = [lax.bitcast_convert_type(x & mask, jnp.float32)
                           for x in xs]
                    return los, his

                acc_lo, acc_hi = split(0)
                for j in range(1, MAX_NB):
                    los, his = split(j)
                    acc_lo = [a + b for a, b in zip(acc_lo, los)]
                    acc_hi = [a + b for a, b in zip(acc_hi, his)]
                for k in range(4):
                    ob[r, pl.ds(k * 16, 16)] = acc_lo[k]
                    ob[r, pl.ds(64 + k * 16, 16)] = acc_hi[k]
                return carry

            lax.fori_loop(0, _RS, row, 0)
            pltpu.async_copy(ob, out.at[pl.ds(base + st * _RS, _RS), :], osem)

        def pair(i, carry):
            halfstep(2 * i, buf0, gsem0, buf1, gsem1, ob0, osem0)
            halfstep(2 * i + 1, buf1, gsem1, buf0, gsem0, ob1, osem1)
            return carry

        lax.fori_loop(0, nsteps // 2, pair, 0)

        pltpu.make_async_copy(ob0, out.at[pl.ds(base, _RS), :], osem0).wait()
        pltpu.make_async_copy(ob1, out.at[pl.ds(base, _RS), :], osem1).wait()

    return pl.kernel(
        body,
        out_type=jax.ShapeDtypeStruct((n_rows, H), jnp.float32),
        mesh=mesh,
        compiler_params=pltpu.CompilerParams(use_tc_tiling_on_sc=False),
        scratch_types=[
            pltpu.VMEM((wlen,), jnp.int32),
            pltpu.VMEM((gwidth, H // 2), jnp.uint32),
            pltpu.VMEM((gwidth, H // 2), jnp.uint32),
            pltpu.VMEM((_RS, H), jnp.float32),
            pltpu.VMEM((_RS, H), jnp.float32),
            pltpu.SemaphoreType.DMA,
            pltpu.SemaphoreType.DMA,
            pltpu.SemaphoreType.DMA,
            pltpu.SemaphoreType.DMA,
        ],
    )


def _pack_half(y):
    u_lo = lax.bitcast_convert_type(y[:, :H // 2], jnp.uint32)
    u_hi = lax.bitcast_convert_type(y[:, H // 2:], jnp.uint32)
    r_lo = u_lo + ((u_lo >> 16) & jnp.uint32(1)) + jnp.uint32(0x7FFF)
    r_hi = u_hi + ((u_hi >> 16) & jnp.uint32(1)) + jnp.uint32(0x7FFF)
    return (r_lo >> 16) | (r_hi & jnp.uint32(0xFFFF0000))


def _pack_rows(y):
    return jnp.concatenate(
        [_pack_half(y[:_BB // 2]), _pack_half(y[_BB // 2:])], axis=1)


def _k0_body(tm_ref, out_ref):
    out_ref[...] = _pack_rows(tm_ref[...])


_k0 = pl.pallas_call(
    _k0_body,
    grid=(TREE_PAD // _BB,),
    in_specs=[pl.BlockSpec((_BB, H), lambda i: (i, 0))],
    out_specs=pl.BlockSpec((_BB // 2, H), lambda i: (i, 0)),
    out_shape=jax.ShapeDtypeStruct((TBL // 2, H), jnp.uint32),
)


def _k1_body(fb_ref, wi_ref, tab_ref, out_ref):
    del tab_ref
    b = jnp.dot(fb_ref[...], wi_ref[...], preferred_element_type=jnp.float32)
    out_ref[...] = _pack_rows(jnp.maximum(b, 0.0))


_k1 = pl.pallas_call(
    _k1_body,
    grid=(NBP // _BB,),
    in_specs=[
        pl.BlockSpec((_BB, ATOM_FDIM + BOND_FDIM), lambda i: (i, 0)),
        pl.BlockSpec((ATOM_FDIM + BOND_FDIM, H), lambda i: (0, 0)),
        pl.BlockSpec(memory_space=pl.ANY),
    ],
    out_specs=pl.BlockSpec((_BB // 2, H),
                           lambda i: (i + TREE_PAD // _BB, 0)),
    out_shape=jax.ShapeDtypeStruct((TBL // 2, H), jnp.uint32),
    input_output_aliases={2: 0},
)


def _k2_body(nei_ref, fb_ref, wi_ref, wh_ref, tab_ref, out_ref):
    del tab_ref
    b = jnp.dot(fb_ref[...], wi_ref[...], preferred_element_type=jnp.float32)
    y = jnp.dot(nei_ref[...], wh_ref[...], preferred_element_type=jnp.float32)
    out_ref[...] = _pack_rows(jnp.maximum(b + y, 0.0))


_k2 = pl.pallas_call(
    _k2_body,
    grid=(NBP // _BB,),
    in_specs=[
        pl.BlockSpec((_BB, H), lambda i: (i, 0)),
        pl.BlockSpec((_BB, ATOM_FDIM + BOND_FDIM), lambda i: (i, 0)),
        pl.BlockSpec((ATOM_FDIM + BOND_FDIM, H), lambda i: (0, 0)),
        pl.BlockSpec((H, H), lambda i: (0, 0)),
        pl.BlockSpec(memory_space=pl.ANY),
    ],
    out_specs=pl.BlockSpec((_BB // 2, H),
                           lambda i: (i + TREE_PAD // _BB, 0)),
    out_shape=jax.ShapeDtypeStruct((TBL // 2, H), jnp.uint32),
    input_output_aliases={4: 0},
)

_MBLK = 40
_ABLK = _MBLK * LE


def _k3_body(an_ref, fa_ref, woa_ref, wob_ref, bo_ref, out_ref):
    h = jnp.dot(fa_ref[...], woa_ref[...], preferred_element_type=jnp.float32)
    h = h + jnp.dot(an_ref[...], wob_ref[...],
                    preferred_element_type=jnp.float32)
    h = jnp.maximum(h + bo_ref[...], 0.0)
    r = lax.broadcasted_iota(jnp.int32, (_MBLK, _ABLK), 1)
    m = lax.broadcasted_iota(jnp.int32, (_MBLK, _ABLK), 0)
    seg = (r // LE == m).astype(jnp.float32)
    out_ref[...] = jnp.dot(seg, h, preferred_element_type=jnp.float32) * (1.0 / LE)


_k3 = pl.pallas_call(
    _k3_body,
    grid=(N_MOLS // _MBLK,),
    in_specs=[
        pl.BlockSpec((_ABLK, H), lambda i: (i, 0)),
        pl.BlockSpec((_ABLK, ATOM_FDIM), lambda i: (i, 0)),
        pl.BlockSpec((ATOM_FDIM, H), lambda i: (0, 0)),
        pl.BlockSpec((H, H), lambda i: (0, 0)),
        pl.BlockSpec((1, H), lambda i: (0, 0)),
    ],
    out_specs=pl.BlockSpec((_MBLK, H), lambda i: (i, 0)),
    out_shape=jax.ShapeDtypeStruct((N_MOLS, H), jnp.float32),
)


def _prep_idx(g, n_pad):
    n = g.shape[0]
    g = g.astype(jnp.int32).reshape(-1)
    g = jnp.where(g >= N_TREE, g + (TREE_PAD - N_TREE), g)
    blk = g >> 10
    off = g & 1023
    g = (blk << 10) + ((off & 511) << 1) + (off >> 9)
    return jnp.pad(g, (0, (n_pad - n) * MAX_NB))


def kernel(fatoms, fbonds, agraph, bgraph, scope, tree_message,
           W_i, W_h, W_o, b_o):
    del scope
    bidx = _prep_idx(bgraph, NBP)
    aidx = _prep_idx(agraph, NAP)
    tree_p = jnp.pad(tree_message, ((0, TREE_PAD - N_TREE), (0, 0)))

    table = _k1(fbonds, W_i, _k0(tree_p))

    for _ in range(DEPTH - 1):
        nei = _make_gather_sum(NBP)(table.reshape(TBL, H // 2), bidx)
        table = _k2(nei, fbonds, W_i, W_h, table)

    anei = _make_gather_sum(NAP)(table.reshape(TBL, H // 2), aidx)
    return _k3(anei, fatoms, W_o[:ATOM_FDIM], W_o[ATOM_FDIM:],
               b_o.reshape(1, H))


# speedup vs baseline: 1.0035x; 1.0035x over previous
import functools

import jax
import jax.numpy as jnp
from jax import lax
from jax.experimental import pallas as pl
from jax.experimental.pallas import tpu as pltpu
from jax.experimental.pallas import tpu_sc as plsc

N_ATOMS = 50000
N_BONDS = 110000
N_TREE = 20000
H = 128
ATOM_FDIM = 35
BOND_FDIM = 5
MAX_NB = 15
DEPTH = 3
N_MOLS = 1000
LE = 50

_NC = 2
_NS = 16
_NW = _NC * _NS
_RS = 8

NBP = 110592
NAP = 50176
TREE_PAD = 20480
TBL = TREE_PAD + NBP
_BB = 1024


@functools.lru_cache(maxsize=None)
def _make_gather_sum(n_rows: int):
    rpw = n_rows // _NW
    nsteps = rpw // _RS
    gwidth = _RS * MAX_NB

    mesh = plsc.VectorSubcoreMesh(
        core_axis_name="c", subcore_axis_name="s",
        num_cores=_NC, num_subcores=_NS)

    def body(table, idxs, out, idx_all, buf0, buf1, ob0, ob1,
             gsem0, gsem1, osem0, osem1):
        cid = lax.axis_index("c")
        sid = lax.axis_index("s")
        wid = sid * _NC + cid
        base = wid * rpw

        pltpu.sync_copy(idxs.at[wid], idx_all)
        pltpu.async_copy(table.at[idx_all.at[0]], buf0, gsem0)

        def halfstep(st, buf, gsem, nbuf, ngsem, ob, osem):
            @pl.when(st + 1 < nsteps)
            def _():
                pltpu.async_copy(table.at[idx_all.at[st + 1]], nbuf, ngsem)

            pltpu.make_async_copy(
                table.at[idx_all.at[st]], buf, gsem).wait()

            @pl.when(st >= 2)
            def _():
                pltpu.make_async_copy(
                    ob, out.at[pl.ds(base, _RS), :], osem).wait()

            def row(r, carry):
                rb = r * MAX_NB
                mask = jnp.uint32(0xFFFF0000)
                sh = jnp.uint32(16)

                def split(j):
                    xs = [buf[rb + j, pl.ds(k * 16, 16)] for k in range(4)]
                    los = [lax.bitcast_convert_type(x << sh, jnp.float32)
                           for x in xs]
                    his = [lax.bitcast_convert_type(x & mask, jnp.float32)
                           for x in xs]
                    return los, his

                acc_lo, acc_hi = split(0)
                for j in range(1, MAX_NB):
                    los, his = split(j)
                    acc_lo = [a + b for a, b in zip(acc_lo, los)]
                    acc_hi = [a + b for a, b in zip(acc_hi, his)]
                for k in range(4):
                    ob[r, pl.ds(k * 16, 16)] = acc_lo[k]
                    ob[r, pl.ds(64 + k * 16, 16)] = acc_hi[k]
                return carry

            lax.fori_loop(0, _RS, row, 0)
            pltpu.async_copy(ob, out.at[pl.ds(base + st * _RS, _RS), :], osem)

        def pair(i, carry):
            halfstep(2 * i, buf0, gsem0, buf1, gsem1, ob0, osem0)
            halfstep(2 * i + 1, buf1, gsem1, buf0, gsem0, ob1, osem1)
            return carry

        lax.fori_loop(0, nsteps // 2, pair, 0)

        pltpu.make_async_copy(ob0, out.at[pl.ds(base, _RS), :], osem0).wait()
        pltpu.make_async_copy(ob1, out.at[pl.ds(base, _RS), :], osem1).wait()

    return pl.kernel(
        body,
        out_type=jax.ShapeDtypeStruct((n_rows, H), jnp.float32),
        mesh=mesh,
        compiler_params=pltpu.CompilerParams(use_tc_tiling_on_sc=False),
        scratch_types=[
            pltpu.VMEM((nsteps, gwidth), jnp.int32),
            pltpu.VMEM((gwidth, H // 2), jnp.uint32),
            pltpu.VMEM((gwidth, H // 2), jnp.uint32),
            pltpu.VMEM((_RS, H), jnp.float32),
            pltpu.VMEM((_RS, H), jnp.float32),
            pltpu.SemaphoreType.DMA,
            pltpu.SemaphoreType.DMA,
            pltpu.SemaphoreType.DMA,
            pltpu.SemaphoreType.DMA,
        ],
    )


def _pack_half(y):
    u_lo = lax.bitcast_convert_type(y[:, :H // 2], jnp.uint32)
    u_hi = lax.bitcast_convert_type(y[:, H // 2:], jnp.uint32)
    r_lo = u_lo + ((u_lo >> 16) & jnp.uint32(1)) + jnp.uint32(0x7FFF)
    r_hi = u_hi + ((u_hi >> 16) & jnp.uint32(1)) + jnp.uint32(0x7FFF)
    return (r_lo >> 16) | (r_hi & jnp.uint32(0xFFFF0000))


def _pack_rows(y):
    return jnp.concatenate(
        [_pack_half(y[:_BB // 2]), _pack_half(y[_BB // 2:])], axis=1)


def _k0_body(tm_ref, out_ref):
    out_ref[...] = _pack_rows(tm_ref[...])


_k0 = pl.pallas_call(
    _k0_body,
    grid=(TREE_PAD // _BB,),
    in_specs=[pl.BlockSpec((_BB, H), lambda i: (i, 0))],
    out_specs=pl.BlockSpec((_BB // 2, H), lambda i: (i, 0)),
    out_shape=jax.ShapeDtypeStruct((TBL // 2, H), jnp.uint32),
)


def _k1_body(fb_ref, wi_ref, tab_ref, out_ref):
    del tab_ref
    b = jnp.dot(fb_ref[...], wi_ref[...], preferred_element_type=jnp.float32)
    out_ref[...] = _pack_rows(jnp.maximum(b, 0.0))


_k1 = pl.pallas_call(
    _k1_body,
    grid=(NBP // _BB,),
    in_specs=[
        pl.BlockSpec((_BB, ATOM_FDIM + BOND_FDIM), lambda i: (i, 0)),
        pl.BlockSpec((ATOM_FDIM + BOND_FDIM, H), lambda i: (0, 0)),
        pl.BlockSpec(memory_space=pl.ANY),
    ],
    out_specs=pl.BlockSpec((_BB // 2, H),
                           lambda i: (i + TREE_PAD // _BB, 0)),
    out_shape=jax.ShapeDtypeStruct((TBL // 2, H), jnp.uint32),
    input_output_aliases={2: 0},
)


def _k2_body(nei_ref, fb_ref, wi_ref, wh_ref, tab_ref, out_ref):
    del tab_ref
    b = jnp.dot(fb_ref[...], wi_ref[...], preferred_element_type=jnp.float32)
    y = jnp.dot(nei_ref[...], wh_ref[...], preferred_element_type=jnp.float32)
    out_ref[...] = _pack_rows(jnp.maximum(b + y, 0.0))


_k2 = pl.pallas_call(
    _k2_body,
    grid=(NBP // _BB,),
    in_specs=[
        pl.BlockSpec((_BB, H), lambda i: (i, 0)),
        pl.BlockSpec((_BB, ATOM_FDIM + BOND_FDIM), lambda i: (i, 0)),
        pl.BlockSpec((ATOM_FDIM + BOND_FDIM, H), lambda i: (0, 0)),
        pl.BlockSpec((H, H), lambda i: (0, 0)),
        pl.BlockSpec(memory_space=pl.ANY),
    ],
    out_specs=pl.BlockSpec((_BB // 2, H),
                           lambda i: (i + TREE_PAD // _BB, 0)),
    out_shape=jax.ShapeDtypeStruct((TBL // 2, H), jnp.uint32),
    input_output_aliases={4: 0},
)

_MBLK = 40
_ABLK = _MBLK * LE


def _k3_body(an_ref, fa_ref, woa_ref, wob_ref, bo_ref, out_ref):
    h = jnp.dot(fa_ref[...], woa_ref[...], preferred_element_type=jnp.float32)
    h = h + jnp.dot(an_ref[...], wob_ref[...],
                    preferred_element_type=jnp.float32)
    h = jnp.maximum(h + bo_ref[...], 0.0)
    r = lax.broadcasted_iota(jnp.int32, (_MBLK, _ABLK), 1)
    m = lax.broadcasted_iota(jnp.int32, (_MBLK, _ABLK), 0)
    seg = (r // LE == m).astype(jnp.float32)
    out_ref[...] = jnp.dot(seg, h, preferred_element_type=jnp.float32) * (1.0 / LE)


_k3 = pl.pallas_call(
    _k3_body,
    grid=(N_MOLS // _MBLK,),
    in_specs=[
        pl.BlockSpec((_ABLK, H), lambda i: (i, 0)),
        pl.BlockSpec((_ABLK, ATOM_FDIM), lambda i: (i, 0)),
        pl.BlockSpec((ATOM_FDIM, H), lambda i: (0, 0)),
        pl.BlockSpec((H, H), lambda i: (0, 0)),
        pl.BlockSpec((1, H), lambda i: (0, 0)),
    ],
    out_specs=pl.BlockSpec((_MBLK, H), lambda i: (i, 0)),
    out_shape=jax.ShapeDtypeStruct((N_MOLS, H), jnp.float32),
)


def _prep_idx(g, n_pad):
    n = g.shape[0]
    g = g.astype(jnp.int32).reshape(-1)
    g = jnp.where(g >= N_TREE, g + (TREE_PAD - N_TREE), g)
    blk = g >> 10
    off = g & 1023
    g = (blk << 10) + ((off & 511) << 1) + (off >> 9)
    g = jnp.pad(g, (0, (n_pad - n) * MAX_NB))
    return g.reshape(_NW, -1, _RS * MAX_NB)


def kernel(fatoms, fbonds, agraph, bgraph, scope, tree_message,
           W_i, W_h, W_o, b_o):
    del scope
    bidx = _prep_idx(bgraph, NBP)
    aidx = _prep_idx(agraph, NAP)
    tree_p = jnp.pad(tree_message, ((0, TREE_PAD - N_TREE), (0, 0)))

    table = _k1(fbonds, W_i, _k0(tree_p))

    for _ in range(DEPTH - 1):
        nei = _make_gather_sum(NBP)(table.reshape(TBL, H // 2), bidx)
        table = _k2(nei, fbonds, W_i, W_h, table)

    anei = _make_gather_sum(NAP)(table.reshape(TBL, H // 2), aidx)
    return _k3(anei, fatoms, W_o[:ATOM_FDIM], W_o[ATOM_FDIM:],
               b_o.reshape(1, H))


# speedup vs baseline: 1.0069x; 1.0034x over previous
import functools

import jax
import jax.numpy as jnp
from jax import lax
from jax.experimental import pallas as pl
from jax.experimental.pallas import tpu as pltpu
from jax.experimental.pallas import tpu_sc as plsc

N_ATOMS = 50000
N_BONDS = 110000
N_TREE = 20000
H = 128
ATOM_FDIM = 35
BOND_FDIM = 5
MAX_NB = 15
DEPTH = 3
N_MOLS = 1000
LE = 50

_NC = 2
_NS = 16
_NW = _NC * _NS
_RS = 8

NBP = 110592
NAP = 50176
TREE_PAD = 20480
TBL = TREE_PAD + NBP
_BB = 1024
_BRP0, _BRP1 = 3984, 2928
_ARP0, _ARP1 = 1808, 1328


@functools.lru_cache(maxsize=None)
def _make_gather_sum(n_rows: int, rp0: int, rp1: int):
    assert _NS * (rp0 + rp1) == n_rows
    ns0, ns1 = rp0 // _RS, rp1 // _RS
    assert ns0 % 2 == 0 and ns1 % 2 == 0
    gwidth = _RS * MAX_NB

    mesh = plsc.VectorSubcoreMesh(
        core_axis_name="c", subcore_axis_name="s",
        num_cores=_NC, num_subcores=_NS)

    def body(table, idxs, out, idx_all, buf0, buf1, ob0, ob1,
             gsem0, gsem1, osem0, osem1):
        cid = lax.axis_index("c")
        sid = lax.axis_index("s")
        base = jnp.where(cid == 0, sid * rp0, _NS * rp0 + sid * rp1)
        nsteps = jnp.where(cid == 0, ns0, ns1)

        @pl.when(cid == 0)
        def _():
            pltpu.sync_copy(idxs.at[pl.ds(base * MAX_NB, rp0 * MAX_NB)],
                            idx_all.at[pl.ds(0, rp0 * MAX_NB)])

        @pl.when(cid != 0)
        def _():
            pltpu.sync_copy(idxs.at[pl.ds(base * MAX_NB, rp1 * MAX_NB)],
                            idx_all.at[pl.ds(0, rp1 * MAX_NB)])

        pltpu.async_copy(table.at[idx_all.at[pl.ds(0, gwidth)]], buf0, gsem0)

        def halfstep(st, buf, gsem, nbuf, ngsem, ob, osem):
            @pl.when(st + 1 < nsteps)
            def _():
                pltpu.async_copy(
                    table.at[idx_all.at[pl.ds((st + 1) * gwidth, gwidth)]],
                    nbuf, ngsem)

            pltpu.make_async_copy(
                table.at[idx_all.at[pl.ds(st * gwidth, gwidth)]],
                buf, gsem).wait()

            @pl.when(st >= 2)
            def _():
                pltpu.make_async_copy(
                    ob, out.at[pl.ds(base, _RS), :], osem).wait()

            def row(r, carry):
                rb = r * MAX_NB
                mask = jnp.uint32(0xFFFF0000)
                sh = jnp.uint32(16)

                def split(j):
                    xs = [buf[rb + j, pl.ds(k * 16, 16)] for k in range(4)]
                    los = [lax.bitcast_convert_type(x << sh, jnp.float32)
                           for x in xs]
                    his = [lax.bitcast_convert_type(x & mask, jnp.float32)
                           for x in xs]
                    return los, his

                acc_lo, acc_hi = split(0)
                for j in range(1, MAX_NB):
                    los, his = split(j)
                    acc_lo = [a + b for a, b in zip(acc_lo, los)]
                    acc_hi = [a + b for a, b in zip(acc_hi, his)]
                for k in range(4):
                    ob[r, pl.ds(k * 16, 16)] = acc_lo[k]
                    ob[r, pl.ds(64 + k * 16, 16)] = acc_hi[k]
                return carry

            lax.fori_loop(0, _RS, row, 0)
            pltpu.async_copy(ob, out.at[pl.ds(base + st * _RS, _RS), :], osem)

        def pair(i, carry):
            halfstep(2 * i, buf0, gsem0, buf1, gsem1, ob0, osem0)
            halfstep(2 * i + 1, buf1, gsem1, buf0, gsem0, ob1, osem1)
            return carry

        lax.fori_loop(0, nsteps // 2, pair, 0)

        pltpu.make_async_copy(ob0, out.at[pl.ds(base, _RS), :], osem0).wait()
        pltpu.make_async_copy(ob1, out.at[pl.ds(base, _RS), :], osem1).wait()

    return pl.kernel(
        body,
        out_type=jax.ShapeDtypeStruct((n_rows, H), jnp.float32),
        mesh=mesh,
        compiler_params=pltpu.CompilerParams(use_tc_tiling_on_sc=False),
        scratch_types=[
            pltpu.VMEM((max(rp0, rp1) * MAX_NB,), jnp.int32),
            pltpu.VMEM((gwidth, H // 2), jnp.uint32),
            pltpu.VMEM((gwidth, H // 2), jnp.uint32),
            pltpu.VMEM((_RS, H), jnp.float32),
            pltpu.VMEM((_RS, H), jnp.float32),
            pltpu.SemaphoreType.DMA,
            pltpu.SemaphoreType.DMA,
            pltpu.SemaphoreType.DMA,
            pltpu.SemaphoreType.DMA,
        ],
    )


def _pack_half(y):
    u_lo = lax.bitcast_convert_type(y[:, :H // 2], jnp.uint32)
    u_hi = lax.bitcast_convert_type(y[:, H // 2:], jnp.uint32)
    r_lo = u_lo + ((u_lo >> 16) & jnp.uint32(1)) + jnp.uint32(0x7FFF)
    r_hi = u_hi + ((u_hi >> 16) & jnp.uint32(1)) + jnp.uint32(0x7FFF)
    return (r_lo >> 16) | (r_hi & jnp.uint32(0xFFFF0000))


def _pack_rows(y):
    return jnp.concatenate(
        [_pack_half(y[:_BB // 2]), _pack_half(y[_BB // 2:])], axis=1)


def _k0_body(tm_ref, out_ref):
    out_ref[...] = _pack_rows(tm_ref[...])


_k0 = pl.pallas_call(
    _k0_body,
    grid=(TREE_PAD // _BB,),
    in_specs=[pl.BlockSpec((_BB, H), lambda i: (i, 0))],
    out_specs=pl.BlockSpec((_BB // 2, H), lambda i: (i, 0)),
    out_shape=jax.ShapeDtypeStruct((TBL // 2, H), jnp.uint32),
)


def _k1_body(fb_ref, wi_ref, tab_ref, out_ref):
    del tab_ref
    b = jnp.dot(fb_ref[...], wi_ref[...], preferred_element_type=jnp.float32)
    out_ref[...] = _pack_rows(jnp.maximum(b, 0.0))


_k1 = pl.pallas_call(
    _k1_body,
    grid=(NBP // _BB,),
    in_specs=[
        pl.BlockSpec((_BB, ATOM_FDIM + BOND_FDIM), lambda i: (i, 0)),
        pl.BlockSpec((ATOM_FDIM + BOND_FDIM, H), lambda i: (0, 0)),
        pl.BlockSpec(memory_space=pl.ANY),
    ],
    out_specs=pl.BlockSpec((_BB // 2, H),
                           lambda i: (i + TREE_PAD // _BB, 0)),
    out_shape=jax.ShapeDtypeStruct((TBL // 2, H), jnp.uint32),
    input_output_aliases={2: 0},
)


def _k2_body(nei_ref, fb_ref, wi_ref, wh_ref, tab_ref, out_ref):
    del tab_ref
    b = jnp.dot(fb_ref[...], wi_ref[...], preferred_element_type=jnp.float32)
    y = jnp.dot(nei_ref[...], wh_ref[...], preferred_element_type=jnp.float32)
    out_ref[...] = _pack_rows(jnp.maximum(b + y, 0.0))


_k2 = pl.pallas_call(
    _k2_body,
    grid=(NBP // _BB,),
    in_specs=[
        pl.BlockSpec((_BB, H), lambda i: (i, 0)),
        pl.BlockSpec((_BB, ATOM_FDIM + BOND_FDIM), lambda i: (i, 0)),
        pl.BlockSpec((ATOM_FDIM + BOND_FDIM, H), lambda i: (0, 0)),
        pl.BlockSpec((H, H), lambda i: (0, 0)),
        pl.BlockSpec(memory_space=pl.ANY),
    ],
    out_specs=pl.BlockSpec((_BB // 2, H),
                           lambda i: (i + TREE_PAD // _BB, 0)),
    out_shape=jax.ShapeDtypeStruct((TBL // 2, H), jnp.uint32),
    input_output_aliases={4: 0},
)

_MBLK = 40
_ABLK = _MBLK * LE


def _k3_body(an_ref, fa_ref, woa_ref, wob_ref, bo_ref, out_ref):
    h = jnp.dot(fa_ref[...], woa_ref[...], preferred_element_type=jnp.float32)
    h = h + jnp.dot(an_ref[...], wob_ref[...],
                    preferred_element_type=jnp.float32)
    h = jnp.maximum(h + bo_ref[...], 0.0)
    r = lax.broadcasted_iota(jnp.int32, (_MBLK, _ABLK), 1)
    m = lax.broadcasted_iota(jnp.int32, (_MBLK, _ABLK), 0)
    seg = (r // LE == m).astype(jnp.float32)
    out_ref[...] = jnp.dot(seg, h, preferred_element_type=jnp.float32) * (1.0 / LE)


_k3 = pl.pallas_call(
    _k3_body,
    grid=(N_MOLS // _MBLK,),
    in_specs=[
        pl.BlockSpec((_ABLK, H), lambda i: (i, 0)),
        pl.BlockSpec((_ABLK, ATOM_FDIM), lambda i: (i, 0)),
        pl.BlockSpec((ATOM_FDIM, H), lambda i: (0, 0)),
        pl.BlockSpec((H, H), lambda i: (0, 0)),
        pl.BlockSpec((1, H), lambda i: (0, 0)),
    ],
    out_specs=pl.BlockSpec((_MBLK, H), lambda i: (i, 0)),
    out_shape=jax.ShapeDtypeStruct((N_MOLS, H), jnp.float32),
)


def _prep_idx(g, n_pad):
    n = g.shape[0]
    g = g.astype(jnp.int32).reshape(-1)
    g = jnp.where(g >= N_TREE, g + (TREE_PAD - N_TREE), g)
    blk = g >> 10
    off = g & 1023
    g = (blk << 10) + ((off & 511) << 1) + (off >> 9)
    return jnp.pad(g, (0, (n_pad - n) * MAX_NB))


def kernel(fatoms, fbonds, agraph, bgraph, scope, tree_message,
           W_i, W_h, W_o, b_o):
    del scope
    bidx = _prep_idx(bgraph, NBP)
    aidx = _prep_idx(agraph, NAP)
    tree_p = jnp.pad(tree_message, ((0, TREE_PAD - N_TREE), (0, 0)))

    table = _k1(fbonds, W_i, _k0(tree_p))

    for _ in range(DEPTH - 1):
        nei = _make_gather_sum(NBP, _BRP0, _BRP1)(
            table.reshape(TBL, H // 2), bidx)
        table = _k2(nei, fbonds, W_i, W_h, table)

    anei = _make_gather_sum(NAP, _ARP0, _ARP1)(
        table.reshape(TBL, H // 2), aidx)
    return _k3(anei, fatoms, W_o[:ATOM_FDIM], W_o[ATOM_FDIM:],
               b_o.reshape(1, H))
